# baseline (speedup 1.0000x reference)
import functools

import jax
import jax.numpy as jnp
from jax import lax
from jax.experimental import pallas as pl
from jax.experimental.pallas import tpu as pltpu
from jax.experimental.pallas import tpu_sc as plsc

D = 128
BLK = 512
NC, NS, L = 2, 16, 16
NW = NC * NS

N0, N1, N2 = 100000, 25000, 6250
PADT = 1 << 20

T1 = 800
WT1 = NW * T1
NP1 = 7168
U1, CH1, NCH1 = 224, 112, 2

T0 = 3200
WT0 = NW * T0
NP0 = 25600
U0, CH0, NCH0 = 800, 80, 10

P1 = 25088
P0 = 100352
PG1 = NP0
PG2 = NP1


def _dotbf(x, w):
    return jnp.dot(
        x.astype(jnp.bfloat16),
        w.astype(jnp.bfloat16),
        preferred_element_type=jnp.float32,
    )


def _affine_body(x_ref, w_ref, b_ref, o_ref):
    o_ref[...] = _dotbf(x_ref[...], w_ref[...]) + b_ref[...]


def _level2_body(x_ref, w_ref, b_ref, wt_ref, h_ref, g_ref):
    h = _dotbf(x_ref[...], w_ref[...]) + b_ref[...]
    h_ref[...] = h
    g_ref[...] = _dotbf(h, wt_ref[...])


def _prep_body(w_ref, b_ref, wb_ref, bup_ref, wc_ref, bc_ref):
    wb = wb_ref[...]
    wc_ref[...] = w_ref[...] + jnp.dot(
        w_ref[...], wb, preferred_element_type=jnp.float32
    )
    bc_ref[...] = (
        b_ref[...]
        + jnp.dot(b_ref[...], wb, preferred_element_type=jnp.float32)
        + bup_ref[...]
    )


def _affine(x, W, b2d, n_out):
    xb = pl.cdiv(x.shape[0], BLK)
    grid = (n_out // BLK,)
    return pl.pallas_call(
        _affine_body,
        grid=grid,
        in_specs=[
            pl.BlockSpec((BLK, D), lambda i: (jnp.minimum(i, xb - 1), 0)),
            pl.BlockSpec((D, D), lambda i: (0, 0)),
            pl.BlockSpec((1, D), lambda i: (0, 0)),
        ],
        out_specs=pl.BlockSpec((BLK, D), lambda i: (i, 0)),
        out_shape=jax.ShapeDtypeStruct((n_out, D), jnp.float32),
    )(x, W, b2d)


def _level2(hn2, W2, b2d, Wt1, n_out):
    xb = pl.cdiv(hn2.shape[0], BLK)
    grid = (n_out // BLK,)
    return pl.pallas_call(
        _level2_body,
        grid=grid,
        in_specs=[
            pl.BlockSpec((BLK, D), lambda i: (jnp.minimum(i, xb - 1), 0)),
            pl.BlockSpec((D, D), lambda i: (0, 0)),
            pl.BlockSpec((1, D), lambda i: (0, 0)),
            pl.BlockSpec((D, D), lambda i: (0, 0)),
        ],
        out_specs=[
            pl.BlockSpec((BLK, D), lambda i: (i, 0)),
            pl.BlockSpec((BLK, D), lambda i: (i, 0)),
        ],
        out_shape=[
            jax.ShapeDtypeStruct((n_out, D), jnp.float32),
            jax.ShapeDtypeStruct((n_out, D), jnp.float32),
        ],
    )(hn2, W2, b2d, Wt1)


def _g1_body(x_ref, w_ref, o_ref, c_ref):
    x = x_ref[...]
    o_ref[...] = _dotbf(x, w_ref[...])
    c_ref[...] = x


def _g1_and_copy(h1f, Wt0):
    xb = pl.cdiv(h1f.shape[0], BLK)
    grid = (PG1 // BLK,)
    return pl.pallas_call(
        _g1_body,
        grid=grid,
        in_specs=[
            pl.BlockSpec((BLK, D), lambda i: (jnp.minimum(i, xb - 1), 0)),
            pl.BlockSpec((D, D), lambda i: (0, 0)),
        ],
        out_specs=[
            pl.BlockSpec((BLK, D), lambda i: (i, 0)),
            pl.BlockSpec((BLK, D), lambda i: (jnp.minimum(i, xb - 1), 0)),
        ],
        out_shape=[
            jax.ShapeDtypeStruct((PG1, D), jnp.float32),
            jax.ShapeDtypeStruct((N1, D), jnp.float32),
        ],
    )(h1f, Wt0)


def _prep(W, b2d, Wb, bup2d):
    return pl.pallas_call(
        _prep_body,
        out_shape=[
            jax.ShapeDtypeStruct((D, D), jnp.float32),
            jax.ShapeDtypeStruct((1, D), jnp.float32),
        ],
    )(W, b2d, Wb, bup2d)


_MESH = plsc.VectorSubcoreMesh(core_axis_name="c", subcore_axis_name="s")


def _wid():
    return lax.axis_index("s") * NC + lax.axis_index("c")


def _make_scan(np_, t_):
    assert (np_ // L) % 2 == 0

    @functools.partial(
        pl.kernel,
        out_type=jax.ShapeDtypeStruct((NW * t_,), jnp.int32),
        mesh=_MESH,
        compiler_params=pltpu.CompilerParams(needs_layout_passes=False),
        scratch_types=[
            pltpu.VMEM((np_,), jnp.int32),
            pltpu.VMEM((t_ + L,), jnp.int32),
            pltpu.VMEM((4 * L,), jnp.int32),
        ],
    )
    def scan_kernel(idx_hbm, win_hbm, idxv, table, tmps):
        wid = _wid()
        iota = lax.iota(jnp.int32, L)
        base = wid * t_
        sent = jnp.full((L,), jnp.int32(2**31 - 1))
        tmps[pl.ds(L, L)] = sent
        tmps[pl.ds(3 * L, L)] = sent

        def initb(k, _):
            table[pl.ds(k * L, L)] = jnp.full((L,), -1, jnp.int32)
            return 0

        lax.fori_loop(0, (t_ + L) // L, initb, 0)
        pltpu.sync_copy(idx_hbm, idxv)

        def chunk(kk, toff):
            t16 = idxv[pl.ds(kk * L, L)]
            loc = t16 - base
            inb = (loc >= 0) & (loc < t_)
            locc = jnp.where(inb, loc, t_)
            comp = (locc << 4) | iota
            skey, _ = plsc.sort_key_val(comp, comp)
            tmps[pl.ds(toff, L)] = skey
            nxt = tmps[pl.ds(toff + 1, L)]
            tgt = skey >> 4
            runlast = (tgt != (nxt >> 4)) | (iota == L - 1)
            mask = runlast & (tgt < t_)
            plsc.store_scatter(table, [tgt], (skey & (L - 1)) + kk * L, mask=mask)

        def body(k, _):
            chunk(2 * k, 0)
            chunk(2 * k + 1, 2 * L)
            return 0

        lax.fori_loop(0, np_ // (2 * L), body, 0)
        pltpu.sync_copy(table.at[pl.ds(0, t_)], win_hbm.at[pl.ds(base, t_)])

    return scan_kernel


_scan1 = _make_scan(NP1, T1)
_scan0 = _make_scan(NP0, T0)


def _make_apply(nb, wt, nreal, u, ch, nch):
    nch16 = ch // L

    @functools.partial(
        pl.kernel,
        out_type=(),
        mesh=_MESH,
        compiler_params=pltpu.CompilerParams(needs_layout_passes=False),
        scratch_types=[
            pltpu.VMEM((u,), jnp.int32),
            pltpu.VMEM((nch, ch), jnp.int32),
            pltpu.VMEM((nch, ch), jnp.int32),
            pltpu.VMEM((nch, ch), jnp.int32),
            pltpu.VMEM((ch, D), jnp.float32),
            pltpu.VMEM((ch, D), jnp.float32),
            pltpu.VMEM((ch, D), jnp.float32),
            pltpu.VMEM((ch, D), jnp.float32),
            pltpu.SemaphoreType.DMA,
            pltpu.SemaphoreType.DMA,
        ],
    )
    def apply_kernel(g_hbm, idxp_hbm, win_hbm, base_ref, traw, tcl, winv, tfin,
                     grow0, grow1, brow0, brow1, gsem, ssem):
        wid = _wid()
        iota = lax.iota(jnp.int32, L)
        wu = wid * u
        grows, brows = (grow0, grow1), (brow0, brow1)
        pltpu.sync_copy(idxp_hbm.at[pl.ds(wu, u)], traw)
        for k in range(u // L):
            r, c = k // nch16, (k % nch16) * L
            t16 = traw[pl.ds(k * L, L)]
            tcl[r, pl.ds(c, L)] = jnp.minimum(t16, wt - 1)
        wdescs = [
            pltpu.async_copy(win_hbm.at[tcl.at[j]], winv.at[j], gsem)
            for j in range(nch)
        ]
        for d in wdescs:
            d.wait()
        for k in range(u // L):
            r, c = k // nch16, (k % nch16) * L
            t16 = traw[pl.ds(k * L, L)]
            wv = winv[r, pl.ds(c, L)]
            iv = iota + (k * L)
            keep = (wv == (iv + wu)) & ((iv + wu) < nreal)
            tfin[r, pl.ds(c, L)] = jnp.where(keep, t16, nb + iota)

        def start_gathers(j):
            b = j & 1
            return (
                pltpu.async_copy(g_hbm.at[pl.ds(wu + j * ch, ch)], grows[b], gsem),
                pltpu.async_copy(base_ref.at[tfin.at[j]], brows[b], gsem),
            )

        pend = start_gathers(0)
        sdescs = [None] * nch
        for j in range(nch):
            b = j & 1
            if j >= 1:
                sdescs[j - 1].wait()
            nxt = start_gathers(j + 1) if j + 1 < nch else None
            pend[0].wait()
            pend[1].wait()
            gr, br = grows[b], brows[b]

            def addb(rr, _, gr=gr, br=br):
                for cc in range(D // L):
                    br[rr, pl.ds(cc * L, L)] = (
                        br[rr, pl.ds(cc * L, L)] + gr[rr, pl.ds(cc * L, L)]
                    )
                return 0

            lax.fori_loop(0, ch, addb, 0)
            sdescs[j] = pltpu.async_copy(br, base_ref.at[tfin.at[j]], ssem)
            pend = nxt
        sdescs[nch - 1].wait()

    return apply_kernel


_apply1 = _make_apply(N1, WT1, N2, U1, CH1, NCH1)
_apply0 = _make_apply(N0, WT0, N1, U0, CH0, NCH0)


def kernel(hn0, hn1, hn2, idx1, idx2, W0, b0, W1, b1, W2, b2, Wup0, bup0, Wup1, bup1):
    b0r, b1r, b2r = b0.reshape(1, D), b1.reshape(1, D), b2.reshape(1, D)
    bup0r, bup1r = bup0.reshape(1, D), bup1.reshape(1, D)
    Wt0, Wb0 = Wup0[:D], Wup0[D:]
    Wt1, Wb1 = Wup1[:D], Wup1[D:]
    zb = jnp.zeros((1, D), jnp.float32)

    idx2p = jnp.pad(idx2, (0, NP1 - N2), constant_values=PADT)
    idx1p = jnp.pad(idx1, (0, NP0 - N1), constant_values=PADT)

    win1 = _scan1(idx2p)
    win0 = _scan0(idx1p)

    Wc0, bc0 = _prep(W0, b0r, Wb0, bup0r)
    Wc1, bc1 = _prep(W1, b1r, Wb1, bup1r)

    h2f, g2 = _level2(hn2, W2, b2r, Wt1, PG2)
    h2 = h2f[:N2]

    h1b = _affine(hn1, Wc1, bc1, P1)
    r1 = jax.new_ref(h1b)
    _apply1(g2, idx2p, win1, r1)
    h1f = r1[...]

    g1, h1_new = _g1_and_copy(h1f, Wt0)

    h0b = _affine(hn0, Wc0, bc0, P0)
    r0 = jax.new_ref(h0b)
    _apply0(g1, idx1p, win0, r0)
    h0_new = r0[...][:N0]

    return (h0_new, h1_new, h2)


# speedup vs baseline: 1.5205x; 1.5205x over previous
import functools

import jax
import jax.numpy as jnp
from jax import lax
from jax.experimental import pallas as pl
from jax.experimental.pallas import tpu as pltpu
from jax.experimental.pallas import tpu_sc as plsc

D = 128
BLK = 2048
NC, NS, L = 2, 16, 16
NW = NC * NS

N0, N1, N2 = 100000, 25000, 6250
PADT = 1 << 20

T1 = 800
WT1 = NW * T1
NP1 = 7168
U1, CH1, NCH1 = 224, 112, 2

T0 = 3200
WT0 = NW * T0
NP0 = 25600
U0, CH0, NCH0 = 800, 80, 10

P1 = 25088
P0 = 100352
PG1 = NP0
PG2 = NP1


def _dotbf(x, w):
    return jnp.dot(
        x.astype(jnp.bfloat16),
        w.astype(jnp.bfloat16),
        preferred_element_type=jnp.float32,
    )


def _affine_body(x_ref, w_ref, b_ref, o_ref):
    o_ref[...] = _dotbf(x_ref[...], w_ref[...]) + b_ref[...]


def _level2_body(x_ref, w_ref, b_ref, wt_ref, h_ref, g_ref):
    h = _dotbf(x_ref[...], w_ref[...]) + b_ref[...]
    h_ref[...] = h
    g_ref[...] = _dotbf(h, wt_ref[...])


def _prep_body(w_ref, b_ref, wb_ref, bup_ref, wc_ref, bc_ref):
    wb = wb_ref[...]
    wc_ref[...] = w_ref[...] + jnp.dot(
        w_ref[...], wb, preferred_element_type=jnp.float32
    )
    bc_ref[...] = (
        b_ref[...]
        + jnp.dot(b_ref[...], wb, preferred_element_type=jnp.float32)
        + bup_ref[...]
    )


def _affine(x, W, b2d, n_out):
    xb = pl.cdiv(x.shape[0], BLK)
    grid = (pl.cdiv(n_out, BLK),)
    return pl.pallas_call(
        _affine_body,
        grid=grid,
        in_specs=[
            pl.BlockSpec((BLK, D), lambda i: (jnp.minimum(i, xb - 1), 0)),
            pl.BlockSpec((D, D), lambda i: (0, 0)),
            pl.BlockSpec((1, D), lambda i: (0, 0)),
        ],
        out_specs=pl.BlockSpec((BLK, D), lambda i: (i, 0)),
        out_shape=jax.ShapeDtypeStruct((n_out, D), jnp.float32),
    )(x, W, b2d)


def _level2(hn2, W2, b2d, Wt1, n_out):
    xb = pl.cdiv(hn2.shape[0], BLK)
    grid = (pl.cdiv(n_out, BLK),)
    return pl.pallas_call(
        _level2_body,
        grid=grid,
        in_specs=[
            pl.BlockSpec((BLK, D), lambda i: (jnp.minimum(i, xb - 1), 0)),
            pl.BlockSpec((D, D), lambda i: (0, 0)),
            pl.BlockSpec((1, D), lambda i: (0, 0)),
            pl.BlockSpec((D, D), lambda i: (0, 0)),
        ],
        out_specs=[
            pl.BlockSpec((BLK, D), lambda i: (i, 0)),
            pl.BlockSpec((BLK, D), lambda i: (i, 0)),
        ],
        out_shape=[
            jax.ShapeDtypeStruct((n_out, D), jnp.float32),
            jax.ShapeDtypeStruct((n_out, D), jnp.float32),
        ],
    )(hn2, W2, b2d, Wt1)


def _g1_body(x_ref, w_ref, o_ref, c_ref):
    x = x_ref[...]
    o_ref[...] = _dotbf(x, w_ref[...])
    c_ref[...] = x


def _g1_and_copy(h1f, Wt0):
    xb = pl.cdiv(h1f.shape[0], BLK)
    grid = (pl.cdiv(PG1, BLK),)
    return pl.pallas_call(
        _g1_body,
        grid=grid,
        in_specs=[
            pl.BlockSpec((BLK, D), lambda i: (jnp.minimum(i, xb - 1), 0)),
            pl.BlockSpec((D, D), lambda i: (0, 0)),
        ],
        out_specs=[
            pl.BlockSpec((BLK, D), lambda i: (i, 0)),
            pl.BlockSpec((BLK, D), lambda i: (jnp.minimum(i, xb - 1), 0)),
        ],
        out_shape=[
            jax.ShapeDtypeStruct((PG1, D), jnp.float32),
            jax.ShapeDtypeStruct((N1, D), jnp.float32),
        ],
    )(h1f, Wt0)


def _prep(W, b2d, Wb, bup2d):
    return pl.pallas_call(
        _prep_body,
        out_shape=[
            jax.ShapeDtypeStruct((D, D), jnp.float32),
            jax.ShapeDtypeStruct((1, D), jnp.float32),
        ],
    )(W, b2d, Wb, bup2d)


_MESH = plsc.VectorSubcoreMesh(core_axis_name="c", subcore_axis_name="s")


def _wid():
    return lax.axis_index("s") * NC + lax.axis_index("c")


def _make_scan(np_, t_):
    assert (np_ // L) % 2 == 0

    @functools.partial(
        pl.kernel,
        out_type=jax.ShapeDtypeStruct((NW * t_,), jnp.int32),
        mesh=_MESH,
        compiler_params=pltpu.CompilerParams(needs_layout_passes=False),
        scratch_types=[
            pltpu.VMEM((np_,), jnp.int32),
            pltpu.VMEM((t_ + L,), jnp.int32),
            pltpu.VMEM((4 * L,), jnp.int32),
        ],
    )
    def scan_kernel(idx_hbm, win_hbm, idxv, table, tmps):
        wid = _wid()
        iota = lax.iota(jnp.int32, L)
        base = wid * t_
        sent = jnp.full((L,), jnp.int32(2**31 - 1))
        tmps[pl.ds(L, L)] = sent
        tmps[pl.ds(3 * L, L)] = sent

        def initb(k, _):
            table[pl.ds(k * L, L)] = jnp.full((L,), -1, jnp.int32)
            return 0

        lax.fori_loop(0, (t_ + L) // L, initb, 0)
        pltpu.sync_copy(idx_hbm, idxv)

        def chunk(kk, toff):
            t16 = idxv[pl.ds(kk * L, L)]
            loc = t16 - base
            inb = (loc >= 0) & (loc < t_)
            locc = jnp.where(inb, loc, t_)
            comp = (locc << 4) | iota
            skey, _ = plsc.sort_key_val(comp, comp)
            tmps[pl.ds(toff, L)] = skey
            nxt = tmps[pl.ds(toff + 1, L)]
            tgt = skey >> 4
            runlast = (tgt != (nxt >> 4)) | (iota == L - 1)
            mask = runlast & (tgt < t_)
            plsc.store_scatter(table, [tgt], (skey & (L - 1)) + kk * L, mask=mask)

        def body(k, _):
            chunk(2 * k, 0)
            chunk(2 * k + 1, 2 * L)
            return 0

        lax.fori_loop(0, np_ // (2 * L), body, 0)
        pltpu.sync_copy(table.at[pl.ds(0, t_)], win_hbm.at[pl.ds(base, t_)])

    return scan_kernel


_scan1 = _make_scan(NP1, T1)
_scan0 = _make_scan(NP0, T0)


def _make_apply(nb, wt, nreal, u, ch, nch):
    nch16 = ch // L

    @functools.partial(
        pl.kernel,
        out_type=(),
        mesh=_MESH,
        compiler_params=pltpu.CompilerParams(needs_layout_passes=False),
        scratch_types=[
            pltpu.VMEM((u,), jnp.int32),
            pltpu.VMEM((nch, ch), jnp.int32),
            pltpu.VMEM((nch, ch), jnp.int32),
            pltpu.VMEM((nch, ch), jnp.int32),
            pltpu.VMEM((ch, D), jnp.float32),
            pltpu.VMEM((ch, D), jnp.float32),
            pltpu.VMEM((ch, D), jnp.float32),
            pltpu.VMEM((ch, D), jnp.float32),
            pltpu.SemaphoreType.DMA,
            pltpu.SemaphoreType.DMA,
        ],
    )
    def apply_kernel(g_hbm, idxp_hbm, win_hbm, base_ref, traw, tcl, winv, tfin,
                     grow0, grow1, brow0, brow1, gsem, ssem):
        wid = _wid()
        iota = lax.iota(jnp.int32, L)
        wu = wid * u
        grows, brows = (grow0, grow1), (brow0, brow1)
        pltpu.sync_copy(idxp_hbm.at[pl.ds(wu, u)], traw)
        for k in range(u // L):
            r, c = k // nch16, (k % nch16) * L
            t16 = traw[pl.ds(k * L, L)]
            tcl[r, pl.ds(c, L)] = jnp.minimum(t16, wt - 1)
        wdescs = [
            pltpu.async_copy(win_hbm.at[tcl.at[j]], winv.at[j], gsem)
            for j in range(nch)
        ]
        for d in wdescs:
            d.wait()
        for k in range(u // L):
            r, c = k // nch16, (k % nch16) * L
            t16 = traw[pl.ds(k * L, L)]
            wv = winv[r, pl.ds(c, L)]
            iv = iota + (k * L)
            keep = (wv == (iv + wu)) & ((iv + wu) < nreal)
            tfin[r, pl.ds(c, L)] = jnp.where(keep, t16, nb + iota)

        def start_gathers(j):
            b = j & 1
            return (
                pltpu.async_copy(g_hbm.at[pl.ds(wu + j * ch, ch)], grows[b], gsem),
                pltpu.async_copy(base_ref.at[tfin.at[j]], brows[b], gsem),
            )

        pend = start_gathers(0)
        sdescs = [None] * nch
        for j in range(nch):
            b = j & 1
            if j >= 1:
                sdescs[j - 1].wait()
            nxt = start_gathers(j + 1) if j + 1 < nch else None
            pend[0].wait()
            pend[1].wait()
            gr, br = grows[b], brows[b]

            def addb(rr, _, gr=gr, br=br):
                for cc in range(D // L):
                    br[rr, pl.ds(cc * L, L)] = (
                        br[rr, pl.ds(cc * L, L)] + gr[rr, pl.ds(cc * L, L)]
                    )
                return 0

            lax.fori_loop(0, ch, addb, 0)
            sdescs[j] = pltpu.async_copy(br, base_ref.at[tfin.at[j]], ssem)
            pend = nxt
        sdescs[nch - 1].wait()

    return apply_kernel


_apply1 = _make_apply(N1, WT1, N2, U1, CH1, NCH1)
_apply0 = _make_apply(N0, WT0, N1, U0, CH0, NCH0)


def kernel(hn0, hn1, hn2, idx1, idx2, W0, b0, W1, b1, W2, b2, Wup0, bup0, Wup1, bup1):
    b0r, b1r, b2r = b0.reshape(1, D), b1.reshape(1, D), b2.reshape(1, D)
    bup0r, bup1r = bup0.reshape(1, D), bup1.reshape(1, D)
    Wt0, Wb0 = Wup0[:D], Wup0[D:]
    Wt1, Wb1 = Wup1[:D], Wup1[D:]
    zb = jnp.zeros((1, D), jnp.float32)

    idx2p = jnp.pad(idx2, (0, NP1 - N2), constant_values=PADT)
    idx1p = jnp.pad(idx1, (0, NP0 - N1), constant_values=PADT)

    win1 = _scan1(idx2p)
    win0 = _scan0(idx1p)

    Wc0, bc0 = _prep(W0, b0r, Wb0, bup0r)
    Wc1, bc1 = _prep(W1, b1r, Wb1, bup1r)

    h2f, g2 = _level2(hn2, W2, b2r, Wt1, PG2)
    h2 = h2f[:N2]

    h1b = _affine(hn1, Wc1, bc1, P1)
    r1 = jax.new_ref(h1b)
    _apply1(g2, idx2p, win1, r1)
    h1f = r1[...]

    g1, h1_new = _g1_and_copy(h1f, Wt0)

    h0b = _affine(hn0, Wc0, bc0, P0)
    r0 = jax.new_ref(h0b)
    _apply0(g1, idx1p, win0, r0)
    h0_new = r0[...][:N0]

    return (h0_new, h1_new, h2)


# speedup vs baseline: 1.6214x; 1.0664x over previous
import functools

import jax
import jax.numpy as jnp
from jax import lax
from jax.experimental import pallas as pl
from jax.experimental.pallas import tpu as pltpu
from jax.experimental.pallas import tpu_sc as plsc

D = 128
BLK = 4096
NC, NS, L = 2, 16, 16
NW = NC * NS

N0, N1, N2 = 100000, 25000, 6250
PADT = 1 << 20

T1 = 800
WT1 = NW * T1
NP1 = 7168
U1, CH1, NCH1 = 224, 112, 2

T0 = 3200
WT0 = NW * T0
NP0 = 25600
U0, CH0, NCH0 = 800, 80, 10

P1 = 25088
P0 = 100352
PG1 = NP0
PG2 = NP1


def _dotbf(x, w):
    return jnp.dot(
        x.astype(jnp.bfloat16),
        w.astype(jnp.bfloat16),
        preferred_element_type=jnp.float32,
    )


def _affine_body(x_ref, w_ref, b_ref, o_ref):
    o_ref[...] = _dotbf(x_ref[...], w_ref[...]) + b_ref[...]


def _level2_body(x_ref, w_ref, b_ref, wt_ref, h_ref, g_ref):
    h = _dotbf(x_ref[...], w_ref[...]) + b_ref[...]
    h_ref[...] = h
    g_ref[...] = _dotbf(h, wt_ref[...])


def _prep_body(w_ref, b_ref, wb_ref, bup_ref, wc_ref, bc_ref):
    wb = wb_ref[...]
    wc_ref[...] = w_ref[...] + jnp.dot(
        w_ref[...], wb, preferred_element_type=jnp.float32
    )
    bc_ref[...] = (
        b_ref[...]
        + jnp.dot(b_ref[...], wb, preferred_element_type=jnp.float32)
        + bup_ref[...]
    )


def _affine(x, W, b2d, n_out):
    xb = pl.cdiv(x.shape[0], BLK)
    grid = (pl.cdiv(n_out, BLK),)
    return pl.pallas_call(
        _affine_body,
        grid=grid,
        in_specs=[
            pl.BlockSpec((BLK, D), lambda i: (jnp.minimum(i, xb - 1), 0)),
            pl.BlockSpec((D, D), lambda i: (0, 0)),
            pl.BlockSpec((1, D), lambda i: (0, 0)),
        ],
        out_specs=pl.BlockSpec((BLK, D), lambda i: (i, 0)),
        out_shape=jax.ShapeDtypeStruct((n_out, D), jnp.float32),
    )(x, W, b2d)


def _level2(hn2, W2, b2d, Wt1, n_out):
    xb = pl.cdiv(hn2.shape[0], BLK)
    grid = (pl.cdiv(n_out, BLK),)
    return pl.pallas_call(
        _level2_body,
        grid=grid,
        in_specs=[
            pl.BlockSpec((BLK, D), lambda i: (jnp.minimum(i, xb - 1), 0)),
            pl.BlockSpec((D, D), lambda i: (0, 0)),
            pl.BlockSpec((1, D), lambda i: (0, 0)),
            pl.BlockSpec((D, D), lambda i: (0, 0)),
        ],
        out_specs=[
            pl.BlockSpec((BLK, D), lambda i: (i, 0)),
            pl.BlockSpec((BLK, D), lambda i: (i, 0)),
        ],
        out_shape=[
            jax.ShapeDtypeStruct((n_out, D), jnp.float32),
            jax.ShapeDtypeStruct((n_out, D), jnp.float32),
        ],
    )(hn2, W2, b2d, Wt1)


def _g1_body(x_ref, w_ref, o_ref, c_ref):
    x = x_ref[...]
    o_ref[...] = _dotbf(x, w_ref[...])
    c_ref[...] = x


def _g1_and_copy(h1f, Wt0):
    xb = pl.cdiv(h1f.shape[0], BLK)
    grid = (pl.cdiv(PG1, BLK),)
    return pl.pallas_call(
        _g1_body,
        grid=grid,
        in_specs=[
            pl.BlockSpec((BLK, D), lambda i: (jnp.minimum(i, xb - 1), 0)),
            pl.BlockSpec((D, D), lambda i: (0, 0)),
        ],
        out_specs=[
            pl.BlockSpec((BLK, D), lambda i: (i, 0)),
            pl.BlockSpec((BLK, D), lambda i: (jnp.minimum(i, xb - 1), 0)),
        ],
        out_shape=[
            jax.ShapeDtypeStruct((PG1, D), jnp.float32),
            jax.ShapeDtypeStruct((N1, D), jnp.float32),
        ],
    )(h1f, Wt0)


def _prep(W, b2d, Wb, bup2d):
    return pl.pallas_call(
        _prep_body,
        out_shape=[
            jax.ShapeDtypeStruct((D, D), jnp.float32),
            jax.ShapeDtypeStruct((1, D), jnp.float32),
        ],
    )(W, b2d, Wb, bup2d)


_MESH = plsc.VectorSubcoreMesh(core_axis_name="c", subcore_axis_name="s")


def _wid():
    return lax.axis_index("s") * NC + lax.axis_index("c")


def _make_scan(np_, t_):
    assert (np_ // L) % 2 == 0

    @functools.partial(
        pl.kernel,
        out_type=jax.ShapeDtypeStruct((NW * t_,), jnp.int32),
        mesh=_MESH,
        compiler_params=pltpu.CompilerParams(needs_layout_passes=False),
        scratch_types=[
            pltpu.VMEM((np_,), jnp.int32),
            pltpu.VMEM((t_ + L,), jnp.int32),
            pltpu.VMEM((4 * L,), jnp.int32),
        ],
    )
    def scan_kernel(idx_hbm, win_hbm, idxv, table, tmps):
        wid = _wid()
        iota = lax.iota(jnp.int32, L)
        base = wid * t_
        sent = jnp.full((L,), jnp.int32(2**31 - 1))
        tmps[pl.ds(L, L)] = sent
        tmps[pl.ds(3 * L, L)] = sent

        def initb(k, _):
            table[pl.ds(k * L, L)] = jnp.full((L,), -1, jnp.int32)
            return 0

        lax.fori_loop(0, (t_ + L) // L, initb, 0)
        pltpu.sync_copy(idx_hbm, idxv)

        def chunk(kk, toff):
            t16 = idxv[pl.ds(kk * L, L)]
            loc = t16 - base
            inb = (loc >= 0) & (loc < t_)
            locc = jnp.where(inb, loc, t_)
            comp = (locc << 4) | iota
            skey, _ = plsc.sort_key_val(comp, comp)
            tmps[pl.ds(toff, L)] = skey
            nxt = tmps[pl.ds(toff + 1, L)]
            tgt = skey >> 4
            runlast = (tgt != (nxt >> 4)) | (iota == L - 1)
            mask = runlast & (tgt < t_)
            plsc.store_scatter(table, [tgt], (skey & (L - 1)) + kk * L, mask=mask)

        def body(k, _):
            chunk(2 * k, 0)
            chunk(2 * k + 1, 2 * L)
            return 0

        lax.fori_loop(0, np_ // (2 * L), body, 0)
        pltpu.sync_copy(table.at[pl.ds(0, t_)], win_hbm.at[pl.ds(base, t_)])

    return scan_kernel


_scan1 = _make_scan(NP1, T1)
_scan0 = _make_scan(NP0, T0)


def _make_apply(nb, wt, nreal, u, ch, nch):
    nch16 = ch // L

    @functools.partial(
        pl.kernel,
        out_type=(),
        mesh=_MESH,
        compiler_params=pltpu.CompilerParams(needs_layout_passes=False),
        scratch_types=[
            pltpu.VMEM((u,), jnp.int32),
            pltpu.VMEM((nch, ch), jnp.int32),
            pltpu.VMEM((nch, ch), jnp.int32),
            pltpu.VMEM((nch, ch), jnp.int32),
            pltpu.VMEM((ch, D), jnp.float32),
            pltpu.VMEM((ch, D), jnp.float32),
            pltpu.VMEM((ch, D), jnp.float32),
            pltpu.VMEM((ch, D), jnp.float32),
            pltpu.SemaphoreType.DMA,
            pltpu.SemaphoreType.DMA,
        ],
    )
    def apply_kernel(g_hbm, idxp_hbm, win_hbm, base_ref, traw, tcl, winv, tfin,
                     grow0, grow1, brow0, brow1, gsem, ssem):
        wid = _wid()
        iota = lax.iota(jnp.int32, L)
        wu = wid * u
        grows, brows = (grow0, grow1), (brow0, brow1)
        pltpu.sync_copy(idxp_hbm.at[pl.ds(wu, u)], traw)
        for k in range(u // L):
            r, c = k // nch16, (k % nch16) * L
            t16 = traw[pl.ds(k * L, L)]
            tcl[r, pl.ds(c, L)] = jnp.minimum(t16, wt - 1)
        wdescs = [
            pltpu.async_copy(win_hbm.at[tcl.at[j]], winv.at[j], gsem)
            for j in range(nch)
        ]
        for d in wdescs:
            d.wait()
        for k in range(u // L):
            r, c = k // nch16, (k % nch16) * L
            t16 = traw[pl.ds(k * L, L)]
            wv = winv[r, pl.ds(c, L)]
            iv = iota + (k * L)
            keep = (wv == (iv + wu)) & ((iv + wu) < nreal)
            tfin[r, pl.ds(c, L)] = jnp.where(keep, t16, nb + iota)

        def start_gathers(j):
            b = j & 1
            return (
                pltpu.async_copy(g_hbm.at[pl.ds(wu + j * ch, ch)], grows[b], gsem),
                pltpu.async_copy(base_ref.at[tfin.at[j]], brows[b], gsem),
            )

        pend = start_gathers(0)
        sdescs = [None] * nch
        for j in range(nch):
            b = j & 1
            if j >= 1:
                sdescs[j - 1].wait()
            nxt = start_gathers(j + 1) if j + 1 < nch else None
            pend[0].wait()
            pend[1].wait()
            gr, br = grows[b], brows[b]

            def addb(rr, _, gr=gr, br=br):
                for cc in range(D // L):
                    br[rr, pl.ds(cc * L, L)] = (
                        br[rr, pl.ds(cc * L, L)] + gr[rr, pl.ds(cc * L, L)]
                    )
                return 0

            lax.fori_loop(0, ch, addb, 0)
            sdescs[j] = pltpu.async_copy(br, base_ref.at[tfin.at[j]], ssem)
            pend = nxt
        sdescs[nch - 1].wait()

    return apply_kernel


_apply1 = _make_apply(N1, WT1, N2, U1, CH1, NCH1)
_apply0 = _make_apply(N0, WT0, N1, U0, CH0, NCH0)


def kernel(hn0, hn1, hn2, idx1, idx2, W0, b0, W1, b1, W2, b2, Wup0, bup0, Wup1, bup1):
    b0r, b1r, b2r = b0.reshape(1, D), b1.reshape(1, D), b2.reshape(1, D)
    bup0r, bup1r = bup0.reshape(1, D), bup1.reshape(1, D)
    Wt0, Wb0 = Wup0[:D], Wup0[D:]
    Wt1, Wb1 = Wup1[:D], Wup1[D:]
    zb = jnp.zeros((1, D), jnp.float32)

    idx2p = jnp.pad(idx2, (0, NP1 - N2), constant_values=PADT)
    idx1p = jnp.pad(idx1, (0, NP0 - N1), constant_values=PADT)

    win1 = _scan1(idx2p)
    win0 = _scan0(idx1p)

    Wc0, bc0 = _prep(W0, b0r, Wb0, bup0r)
    Wc1, bc1 = _prep(W1, b1r, Wb1, bup1r)

    h2f, g2 = _level2(hn2, W2, b2r, Wt1, PG2)
    h2 = h2f[:N2]

    h1b = _affine(hn1, Wc1, bc1, P1)
    r1 = jax.new_ref(h1b)
    _apply1(g2, idx2p, win1, r1)
    h1f = r1[...]

    g1, h1_new = _g1_and_copy(h1f, Wt0)

    h0b = _affine(hn0, Wc0, bc0, P0)
    r0 = jax.new_ref(h0b)
    _apply0(g1, idx1p, win0, r0)
    h0_new = r0[...][:N0]

    return (h0_new, h1_new, h2)
